# baseline (speedup 1.0000x reference)
import jax
import jax.numpy as jnp
from jax import lax
from jax.experimental import pallas as pl
from jax.experimental.pallas import tpu as pltpu
from jax.experimental.pallas import tpu_sc as plsc

_VOCAB = 1000000
_MAXLEN = 200
_EMBED = 32
_BATCH = 4096

_NW = 32
_BPW = _BATCH // _NW
_IDXW = _BPW * _MAXLEN
_JB = _EMBED // 8


_NCB = _VOCAB // 128
_TAIL = _VOCAB - _NCB * 128


def _tr_body(tokt_hbm, tail_hbm, out_hbm, tb0, tb1, sk, os0, os1,
             gs0, gs1, ws0, ws1):
    cid = lax.axis_index("c")
    sid = lax.axis_index("s")
    wid = sid * 2 + cid

    iota16 = lax.broadcasted_iota(jnp.int32, (16,), 0)
    zeros16 = jnp.zeros((16,), jnp.int32)
    base_lo = iota16 * 133
    base_hi = (iota16 + 16) * 133

    tb = (tb0, tb1)
    osg = (os0, os1)
    gsem = (gs0, gs1)
    wsem = (ws0, ws1)

    def fire_read(cb, b):
        for jb in range(4):
            pltpu.async_copy(
                tokt_hbm.at[pl.ds(jb * 8, 8), pl.ds(cb * 128, 128)],
                tb[b].at[jb, pl.ds(0, 8), pl.ds(0, 128)], gsem[b])

    def wait_read(b):
        for jb in range(4):
            pltpu.make_async_copy(
                tokt_hbm.at[pl.ds(jb * 8, 8), pl.ds(0, 128)],
                tb[b].at[jb, pl.ds(0, 8), pl.ds(0, 128)], gsem[b]).wait()

    def transpose(b):
        def s1(j, carry):
            jb = j // 8
            r = j - jb * 8
            for c0 in range(8):
                sk[pl.ds(j * 133 + c0 * 16, 16)] = (
                    tb[b][jb, r, pl.ds(c0 * 16, 16)])
            return carry
        lax.fori_loop(0, 32, s1, 0)

        def s2(q, carry):
            for c0 in range(8):
                base = base_lo if c0 % 2 == 0 else base_hi
                v = plsc.load_gather(sk, [base + (4 * q + c0 // 2)])
                osg[b][q, pl.ds(c0 * 16, 16)] = v
            return carry
        lax.fori_loop(0, 32, s2, 0)

    def fire_write(cb, b):
        pltpu.async_copy(osg[b], out_hbm.at[pl.ds(cb * 32, 32), :], wsem[b])

    def wait_write(b):
        pltpu.make_async_copy(osg[b], out_hbm.at[pl.ds(0, 32), :],
                              wsem[b]).wait()

    nt_full = (_NCB - wid + _NW - 1) // _NW
    has_tail = wid == (_NCB % _NW)

    fire_read(wid, 0)

    def outer(t2, carry):
        for b in range(2):
            t = t2 * 2 + b
            cb = wid + t * _NW
            nxt = cb + _NW

            @pl.when(t + 1 < nt_full)
            def _prefetch():
                fire_read(nxt, 1 - b)

            @pl.when(t < nt_full)
            def _work():
                wait_read(b)

                @pl.when(t >= 2)
                def _drain():
                    wait_write(b)

                transpose(b)
                fire_write(cb, b)
        return carry

    lax.fori_loop(0, (nt_full + 1) // 2, outer, 0)

    wait_write(0)
    wait_write(1)

    @pl.when(has_tail)
    def _tail():
        pltpu.sync_copy(tail_hbm, osg[0].at[pl.ds(0, 16), :])
        pltpu.sync_copy(osg[0].at[pl.ds(0, 16), :],
                        out_hbm.at[pl.ds(_NCB * 32, 16), :])


def _sc_body(x_hbm, tok_hbm, pos_hbm, out_hbm,
             x_t, g0, g1, s0, s1, posf, gs0, gs1, os0, os1):
    cid = lax.axis_index("c")
    sid = lax.axis_index("s")
    wid = sid * 2 + cid

    pltpu.sync_copy(x_hbm.at[:, wid], x_t)
    pltpu.sync_copy(pos_hbm, posf)

    iota16 = lax.broadcasted_iota(jnp.int32, (16,), 0)
    zeros16 = jnp.zeros((16,), jnp.int32)

    g = (g0, g1)
    s = (s0, s1)
    gsem = (gs0, gs1)
    osem = (os0, os1)

    def fire_gather(p, b):
        pltpu.async_copy(tok_hbm.at[x_t.at[p // 8, p % 8]], g[b], gsem[b])

    def wait_gather(b):
        pltpu.make_async_copy(tok_hbm.at[x_t.at[0, 0]], g[b], gsem[b]).wait()

    def fire_out(p, b):
        for jb in range(_JB):
            pltpu.async_copy(s[b].at[pl.ds(jb * 8, 8), pl.ds(0, 128)],
                             out_hbm.at[p, jb, wid], osem[b])

    def wait_out(b):
        for jb in range(_JB):
            pltpu.make_async_copy(s[b].at[pl.ds(jb * 8, 8), pl.ds(0, 128)],
                                  out_hbm.at[0, jb, wid], osem[b]).wait()

    row_lo = iota16
    row_hi = iota16 + 16

    def compute(p, b):
        p32 = p * _EMBED
        pos0 = posf[pl.ds(p32, 16)]
        pos1 = posf[pl.ds(p32 + 16, 16)]
        for bi in range(_BPW):
            colv = zeros16 + bi
            v0 = g[b][bi, pl.ds(0, 16)] + pos0
            v1 = g[b][bi, pl.ds(16, 16)] + pos1
            plsc.store_scatter(s[b], [row_lo, colv], v0)
            plsc.store_scatter(s[b], [row_hi, colv], v1)

    fire_gather(0, 0)

    def outer(p2, carry):
        for b in range(2):
            p = p2 * 2 + b
            nxt = p + 1

            @pl.when(nxt < _MAXLEN)
            def _prefetch():
                fire_gather(nxt, 1 - b)

            wait_gather(b)

            @pl.when(p >= 2)
            def _drain():
                wait_out(b)

            compute(p, b)
            fire_out(p, b)
        return carry

    lax.fori_loop(0, _MAXLEN // 2, outer, 0)
    wait_out(0)
    wait_out(1)


def kernel(x, token_table, pos_table):
    x5 = (x.astype(jnp.int32)
          .reshape(_NW, 128, _MAXLEN // 8, 8)
          .transpose(2, 0, 3, 1))
    pos_flat = pos_table.reshape(-1)
    mesh = plsc.VectorSubcoreMesh(core_axis_name="c", subcore_axis_name="s")

    tr = pl.kernel(
        _tr_body,
        out_type=jax.ShapeDtypeStruct((_VOCAB // 4, 128), jnp.float32),
        mesh=mesh,
        compiler_params=pltpu.CompilerParams(use_tc_tiling_on_sc=True,
                                             needs_layout_passes=False),
        scratch_types=[
            pltpu.VMEM((4, 8, 128), jnp.float32),
            pltpu.VMEM((4, 8, 128), jnp.float32),
            pltpu.VMEM((32 * 133,), jnp.float32),
            pltpu.VMEM((32, 128), jnp.float32),
            pltpu.VMEM((32, 128), jnp.float32),
            pltpu.SemaphoreType.DMA,
            pltpu.SemaphoreType.DMA,
            pltpu.SemaphoreType.DMA,
            pltpu.SemaphoreType.DMA,
        ],
    )
    tail16 = token_table[_NCB * 128:].reshape(16, 128)
    table_rm = tr(token_table.T, tail16).reshape(_VOCAB, _EMBED)
    f = pl.kernel(
        _sc_body,
        out_type=jax.ShapeDtypeStruct((_MAXLEN, _JB, _NW, 8, 128),
                                      jnp.float32),
        mesh=mesh,
        compiler_params=pltpu.CompilerParams(use_tc_tiling_on_sc=False,
                                             needs_layout_passes=False),
        scratch_types=[
            pltpu.VMEM((_MAXLEN // 8, 8, _BPW), jnp.int32),
            pltpu.VMEM((_BPW, _EMBED), jnp.float32),
            pltpu.VMEM((_BPW, _EMBED), jnp.float32),
            pltpu.VMEM((_EMBED, 129), jnp.float32),
            pltpu.VMEM((_EMBED, 129), jnp.float32),
            pltpu.VMEM((_MAXLEN * _EMBED,), jnp.float32),
            pltpu.SemaphoreType.DMA,
            pltpu.SemaphoreType.DMA,
            pltpu.SemaphoreType.DMA,
            pltpu.SemaphoreType.DMA,
        ],
    )
    out5 = f(x5, table_rm, pos_flat)
    return out5.transpose(2, 4, 0, 1, 3).reshape(_BATCH, _MAXLEN, _EMBED)


# speedup vs baseline: 1.1090x; 1.1090x over previous
import jax
import jax.numpy as jnp
from jax import lax
from jax.experimental import pallas as pl
from jax.experimental.pallas import tpu as pltpu
from jax.experimental.pallas import tpu_sc as plsc

_VOCAB = 1000000
_MAXLEN = 200
_EMBED = 32
_BATCH = 4096

_NW = 32
_BPW = _BATCH // _NW
_IDXW = _BPW * _MAXLEN
_JB = _EMBED // 8


_NCB = _VOCAB // 128
_TAIL = _VOCAB - _NCB * 128


def _tr_body(tokt_hbm, tail_hbm, out_hbm, tb0, tb1, sk, os0, os1,
             gs0, gs1, ws0, ws1):
    cid = lax.axis_index("c")
    sid = lax.axis_index("s")
    wid = sid * 2 + cid

    iota16 = lax.broadcasted_iota(jnp.int32, (16,), 0)
    zeros16 = jnp.zeros((16,), jnp.int32)
    base_lo = iota16 * 133
    base_hi = (iota16 + 16) * 133

    tb = (tb0, tb1)
    osg = (os0, os1)
    gsem = (gs0, gs1)
    wsem = (ws0, ws1)

    def fire_read(cb, b):
        for jb in range(4):
            pltpu.async_copy(
                tokt_hbm.at[pl.ds(jb * 8, 8), pl.ds(cb * 128, 128)],
                tb[b].at[jb, pl.ds(0, 8), pl.ds(0, 128)], gsem[b])

    def wait_read(b):
        for jb in range(4):
            pltpu.make_async_copy(
                tokt_hbm.at[pl.ds(jb * 8, 8), pl.ds(0, 128)],
                tb[b].at[jb, pl.ds(0, 8), pl.ds(0, 128)], gsem[b]).wait()

    def transpose(b):
        for jb in range(4):
            for r in range(8):
                j = jb * 8 + r
                for c0 in range(8):
                    sk[pl.ds(j * 133 + c0 * 16, 16)] = (
                        tb[b][jb, r, pl.ds(c0 * 16, 16)])
        for q in range(32):
            for c0 in range(8):
                base = base_lo if c0 % 2 == 0 else base_hi
                v = plsc.load_gather(sk, [base + (4 * q + c0 // 2)])
                osg[b][q, pl.ds(c0 * 16, 16)] = v

    def fire_write(cb, b):
        pltpu.async_copy(osg[b], out_hbm.at[pl.ds(cb * 32, 32), :], wsem[b])

    def wait_write(b):
        pltpu.make_async_copy(osg[b], out_hbm.at[pl.ds(0, 32), :],
                              wsem[b]).wait()

    nt_full = (_NCB - wid + _NW - 1) // _NW
    has_tail = wid == (_NCB % _NW)

    fire_read(wid, 0)

    def outer(t2, carry):
        for b in range(2):
            t = t2 * 2 + b
            cb = wid + t * _NW
            nxt = cb + _NW

            @pl.when(t + 1 < nt_full)
            def _prefetch():
                fire_read(nxt, 1 - b)

            @pl.when(t < nt_full)
            def _work():
                wait_read(b)

                @pl.when(t >= 2)
                def _drain():
                    wait_write(b)

                transpose(b)
                fire_write(cb, b)
        return carry

    lax.fori_loop(0, (nt_full + 1) // 2, outer, 0)

    wait_write(0)
    wait_write(1)

    @pl.when(has_tail)
    def _tail():
        pltpu.sync_copy(tail_hbm, osg[0].at[pl.ds(0, 16), :])
        pltpu.sync_copy(osg[0].at[pl.ds(0, 16), :],
                        out_hbm.at[pl.ds(_NCB * 32, 16), :])


def _sc_body(x_hbm, tok_hbm, pos_hbm, out_hbm,
             x_t, g0, g1, s0, s1, posf, gs0, gs1, os0, os1):
    cid = lax.axis_index("c")
    sid = lax.axis_index("s")
    wid = sid * 2 + cid

    pltpu.sync_copy(x_hbm.at[:, wid], x_t)
    pltpu.sync_copy(pos_hbm, posf)

    iota16 = lax.broadcasted_iota(jnp.int32, (16,), 0)
    zeros16 = jnp.zeros((16,), jnp.int32)

    g = (g0, g1)
    s = (s0, s1)
    gsem = (gs0, gs1)
    osem = (os0, os1)

    def fire_gather(p, b):
        pltpu.async_copy(tok_hbm.at[x_t.at[p // 8, p % 8]], g[b], gsem[b])

    def wait_gather(b):
        pltpu.make_async_copy(tok_hbm.at[x_t.at[0, 0]], g[b], gsem[b]).wait()

    def fire_out(p, b):
        for jb in range(_JB):
            pltpu.async_copy(s[b].at[pl.ds(jb * 8, 8), pl.ds(0, 128)],
                             out_hbm.at[p, jb, wid], osem[b])

    def wait_out(b):
        for jb in range(_JB):
            pltpu.make_async_copy(s[b].at[pl.ds(jb * 8, 8), pl.ds(0, 128)],
                                  out_hbm.at[0, jb, wid], osem[b]).wait()

    row_lo = iota16
    row_hi = iota16 + 16

    def compute(p, b):
        p32 = p * _EMBED
        pos0 = posf[pl.ds(p32, 16)]
        pos1 = posf[pl.ds(p32 + 16, 16)]
        for bi in range(_BPW):
            colv = zeros16 + bi
            v0 = g[b][bi, pl.ds(0, 16)] + pos0
            v1 = g[b][bi, pl.ds(16, 16)] + pos1
            plsc.store_scatter(s[b], [row_lo, colv], v0)
            plsc.store_scatter(s[b], [row_hi, colv], v1)

    fire_gather(0, 0)

    def outer(p2, carry):
        for b in range(2):
            p = p2 * 2 + b
            nxt = p + 1

            @pl.when(nxt < _MAXLEN)
            def _prefetch():
                fire_gather(nxt, 1 - b)

            wait_gather(b)

            @pl.when(p >= 2)
            def _drain():
                wait_out(b)

            compute(p, b)
            fire_out(p, b)
        return carry

    lax.fori_loop(0, _MAXLEN // 2, outer, 0)
    wait_out(0)
    wait_out(1)


def kernel(x, token_table, pos_table):
    x5 = (x.astype(jnp.int32)
          .reshape(_NW, 128, _MAXLEN // 8, 8)
          .transpose(2, 0, 3, 1))
    pos_flat = pos_table.reshape(-1)
    mesh = plsc.VectorSubcoreMesh(core_axis_name="c", subcore_axis_name="s")

    tr = pl.kernel(
        _tr_body,
        out_type=jax.ShapeDtypeStruct((_VOCAB // 4, 128), jnp.float32),
        mesh=mesh,
        compiler_params=pltpu.CompilerParams(use_tc_tiling_on_sc=True,
                                             needs_layout_passes=False),
        scratch_types=[
            pltpu.VMEM((4, 8, 128), jnp.float32),
            pltpu.VMEM((4, 8, 128), jnp.float32),
            pltpu.VMEM((32 * 133,), jnp.float32),
            pltpu.VMEM((32, 128), jnp.float32),
            pltpu.VMEM((32, 128), jnp.float32),
            pltpu.SemaphoreType.DMA,
            pltpu.SemaphoreType.DMA,
            pltpu.SemaphoreType.DMA,
            pltpu.SemaphoreType.DMA,
        ],
    )
    tail16 = token_table[_NCB * 128:].reshape(16, 128)
    table_rm = tr(token_table.T, tail16).reshape(_VOCAB, _EMBED)
    f = pl.kernel(
        _sc_body,
        out_type=jax.ShapeDtypeStruct((_MAXLEN, _JB, _NW, 8, 128),
                                      jnp.float32),
        mesh=mesh,
        compiler_params=pltpu.CompilerParams(use_tc_tiling_on_sc=False,
                                             needs_layout_passes=False),
        scratch_types=[
            pltpu.VMEM((_MAXLEN // 8, 8, _BPW), jnp.int32),
            pltpu.VMEM((_BPW, _EMBED), jnp.float32),
            pltpu.VMEM((_BPW, _EMBED), jnp.float32),
            pltpu.VMEM((_EMBED, 129), jnp.float32),
            pltpu.VMEM((_EMBED, 129), jnp.float32),
            pltpu.VMEM((_MAXLEN * _EMBED,), jnp.float32),
            pltpu.SemaphoreType.DMA,
            pltpu.SemaphoreType.DMA,
            pltpu.SemaphoreType.DMA,
            pltpu.SemaphoreType.DMA,
        ],
    )
    out5 = f(x5, table_rm, pos_flat)
    return out5.transpose(2, 4, 0, 1, 3).reshape(_BATCH, _MAXLEN, _EMBED)


# speedup vs baseline: 1.1129x; 1.0036x over previous
import jax
import jax.numpy as jnp
from jax import lax
from jax.experimental import pallas as pl
from jax.experimental.pallas import tpu as pltpu
from jax.experimental.pallas import tpu_sc as plsc

_VOCAB = 1000000
_MAXLEN = 200
_EMBED = 32
_BATCH = 4096

_NW = 32
_BPW = _BATCH // _NW
_JB = _EMBED // 8
_NCB = _VOCAB // 128


def _tr_body(tokt_hbm, tail_hbm, out_hbm, tb0, tb1, sk, os0, os1,
             gs0, gs1, ws0, ws1):
    cid = lax.axis_index("c")
    sid = lax.axis_index("s")
    wid = sid * 2 + cid

    iota16 = lax.broadcasted_iota(jnp.int32, (16,), 0)
    base_lo = iota16 * 133
    base_hi = (iota16 + 16) * 133

    tb = (tb0, tb1)
    osg = (os0, os1)
    gsem = (gs0, gs1)
    wsem = (ws0, ws1)

    def fire_read(cb, b):
        for jb in range(4):
            pltpu.async_copy(
                tokt_hbm.at[pl.ds(jb * 8, 8), pl.ds(cb * 128, 128)],
                tb[b].at[jb, pl.ds(0, 8), pl.ds(0, 128)], gsem[b])

    def wait_read(b):
        for jb in range(4):
            pltpu.make_async_copy(
                tokt_hbm.at[pl.ds(jb * 8, 8), pl.ds(0, 128)],
                tb[b].at[jb, pl.ds(0, 8), pl.ds(0, 128)], gsem[b]).wait()

    def transpose(b):
        for jb in range(4):
            for r in range(8):
                j = jb * 8 + r
                for c0 in range(8):
                    sk[pl.ds(j * 133 + c0 * 16, 16)] = (
                        tb[b][jb, r, pl.ds(c0 * 16, 16)])
        for q in range(32):
            for c0 in range(8):
                base = base_lo if c0 % 2 == 0 else base_hi
                v = plsc.load_gather(sk, [base + (4 * q + c0 // 2)])
                osg[b][q, pl.ds(c0 * 16, 16)] = v

    def fire_write(cb, b):
        pltpu.async_copy(osg[b], out_hbm.at[pl.ds(cb * 32, 32), :], wsem[b])

    def wait_write(b):
        pltpu.make_async_copy(osg[b], out_hbm.at[pl.ds(0, 32), :],
                              wsem[b]).wait()

    nt_full = (_NCB - wid + _NW - 1) // _NW
    has_tail = wid == (_NCB % _NW)

    fire_read(wid, 0)

    def outer(t2, carry):
        for b in range(2):
            t = t2 * 2 + b
            cb = wid + t * _NW
            nxt = cb + _NW

            @pl.when(t + 1 < nt_full)
            def _prefetch():
                fire_read(nxt, 1 - b)

            @pl.when(t < nt_full)
            def _work():
                wait_read(b)

                @pl.when(t >= 2)
                def _drain():
                    wait_write(b)

                transpose(b)
                fire_write(cb, b)
        return carry

    lax.fori_loop(0, (nt_full + 1) // 2, outer, 0)

    wait_write(0)
    wait_write(1)

    @pl.when(has_tail)
    def _tail():
        pltpu.sync_copy(tail_hbm, osg[0].at[pl.ds(0, 16), :])
        pltpu.sync_copy(osg[0].at[pl.ds(0, 16), :],
                        out_hbm.at[pl.ds(_NCB * 32, 16), :])


def _sc_body(x_hbm, tok_hbm, pos_hbm, out_hbm,
             x_t, g0, g1, s0, s1, posf, gs0, gs1, os0, os1):
    cid = lax.axis_index("c")
    sid = lax.axis_index("s")
    wid = sid * 2 + cid

    pltpu.sync_copy(x_hbm.at[:, wid], x_t)
    pltpu.sync_copy(pos_hbm, posf)

    iota16 = lax.broadcasted_iota(jnp.int32, (16,), 0)
    zeros16 = jnp.zeros((16,), jnp.int32)

    g = (g0, g1)
    s = (s0, s1)
    gsem = (gs0, gs1)
    osem = (os0, os1)

    def fire_gather(p, b):
        pltpu.async_copy(tok_hbm.at[x_t.at[p // 8, p % 8]], g[b], gsem[b])

    def wait_gather(b):
        pltpu.make_async_copy(tok_hbm.at[x_t.at[0, 0]], g[b], gsem[b]).wait()

    def fire_out(p, b):
        for jb in range(_JB):
            pltpu.async_copy(s[b].at[pl.ds(jb * 8, 8), pl.ds(0, 128)],
                             out_hbm.at[p, jb, wid], osem[b])

    def wait_out(b):
        for jb in range(_JB):
            pltpu.make_async_copy(s[b].at[pl.ds(jb * 8, 8), pl.ds(0, 128)],
                                  out_hbm.at[0, jb, wid], osem[b]).wait()

    row_lo = iota16
    row_hi = iota16 + 16

    def compute(p, b):
        p32 = p * _EMBED
        pos0 = posf[pl.ds(p32, 16)]
        pos1 = posf[pl.ds(p32 + 16, 16)]
        for bi in range(_BPW):
            colv = zeros16 + bi
            v0 = g[b][bi, pl.ds(0, 16)] + pos0
            v1 = g[b][bi, pl.ds(16, 16)] + pos1
            plsc.store_scatter(s[b], [row_lo, colv], v0)
            plsc.store_scatter(s[b], [row_hi, colv], v1)

    fire_gather(0, 0)

    def outer(p2, carry):
        for b in range(2):
            p = p2 * 2 + b
            nxt = p + 1

            @pl.when(nxt < _MAXLEN)
            def _prefetch():
                fire_gather(nxt, 1 - b)

            wait_gather(b)

            @pl.when(p >= 2)
            def _drain():
                wait_out(b)

            compute(p, b)
            fire_out(p, b)
        return carry

    lax.fori_loop(0, _MAXLEN // 2, outer, 0)
    wait_out(0)
    wait_out(1)


def kernel(x, token_table, pos_table):
    x5 = (x.astype(jnp.int32)
          .reshape(_NW, 128, _MAXLEN // 8, 8)
          .transpose(2, 0, 3, 1))
    pos_flat = pos_table.reshape(-1)
    mesh = plsc.VectorSubcoreMesh(core_axis_name="c", subcore_axis_name="s")

    tr = pl.kernel(
        _tr_body,
        out_type=jax.ShapeDtypeStruct((_VOCAB // 4, 128), jnp.float32),
        mesh=mesh,
        compiler_params=pltpu.CompilerParams(use_tc_tiling_on_sc=True,
                                             needs_layout_passes=False),
        scratch_types=[
            pltpu.VMEM((4, 8, 128), jnp.float32),
            pltpu.VMEM((4, 8, 128), jnp.float32),
            pltpu.VMEM((32 * 133,), jnp.float32),
            pltpu.VMEM((32, 128), jnp.float32),
            pltpu.VMEM((32, 128), jnp.float32),
            pltpu.SemaphoreType.DMA,
            pltpu.SemaphoreType.DMA,
            pltpu.SemaphoreType.DMA,
            pltpu.SemaphoreType.DMA,
        ],
    )
    tail16 = token_table[_NCB * 128:].reshape(16, 128)
    table_rm = tr(token_table.T, tail16).reshape(_VOCAB, _EMBED)
    f = pl.kernel(
        _sc_body,
        out_type=jax.ShapeDtypeStruct((_MAXLEN, _JB, _NW, 8, 128),
                                      jnp.float32),
        mesh=mesh,
        compiler_params=pltpu.CompilerParams(use_tc_tiling_on_sc=False,
                                             needs_layout_passes=False),
        scratch_types=[
            pltpu.VMEM((_MAXLEN // 8, 8, _BPW), jnp.int32),
            pltpu.VMEM((_BPW, _EMBED), jnp.float32),
            pltpu.VMEM((_BPW, _EMBED), jnp.float32),
            pltpu.VMEM((_EMBED, 129), jnp.float32),
            pltpu.VMEM((_EMBED, 129), jnp.float32),
            pltpu.VMEM((_MAXLEN * _EMBED,), jnp.float32),
            pltpu.SemaphoreType.DMA,
            pltpu.SemaphoreType.DMA,
            pltpu.SemaphoreType.DMA,
            pltpu.SemaphoreType.DMA,
        ],
    )
    out5 = f(x5, table_rm, pos_flat)
    return out5.transpose(2, 4, 0, 1, 3).reshape(_BATCH, _MAXLEN, _EMBED)
